# baseline (speedup 1.0000x reference)
import functools

import jax
import jax.numpy as jnp
from jax import lax
from jax.experimental import pallas as pl
from jax.experimental.pallas import tpu as pltpu
from jax.experimental.pallas import tpu_sc as plsc

POOL_H = 7
POOL_W = 7
NSAMP = POOL_H * POOL_W
SPAD = 56
WPAD = 64
NBOX = 1000
NPAD = 1024
NTILES = 32
PER_TILE = NPAD // NTILES
C = 256


OFF3 = 256 * 256
OFF4 = OFF3 + 128 * 128
OFF5 = OFF4 + 64 * 64
TROWS = OFF5 + 32 * 32


def _prelude_body(boxes_ref, imeta_ref, idx_ref, wts_ref):
    b = boxes_ref[...]
    y1 = b[:, 0:1]
    x1 = b[:, 1:2]
    y2 = b[:, 2:3]
    x2 = b[:, 3:4]
    h = y2 - y1
    w = x2 - x1
    m = imeta_ref[...]
    area = m[0, 4] * m[0, 5]
    rl = jnp.log(jnp.sqrt(h * w) / (224.0 / jnp.sqrt(area))) / jnp.log(2.0)
    lvl = jnp.minimum(5, jnp.maximum(2, 4 + jnp.round(rl).astype(jnp.int32)))
    side = jnp.right_shift(1024, lvl)
    off = jnp.where(lvl == 2, 0,
                    jnp.where(lvl == 3, OFF3,
                              jnp.where(lvl == 4, OFF4, OFF5)))
    sm1i = side - 1
    sm1f = sm1i.astype(jnp.float32)

    s = lax.broadcasted_iota(jnp.int32, (1, WPAD), 1)
    iy = (s // POOL_W).astype(jnp.float32)
    ix = (s % POOL_W).astype(jnp.float32)
    ys = y1 * sm1f + iy * (h * sm1f / (POOL_H - 1))
    xs = x1 * sm1f + ix * (w * sm1f / (POOL_W - 1))
    y0f = jnp.floor(ys)
    x0f = jnp.floor(xs)
    y0 = jnp.clip(y0f.astype(jnp.int32), 0, sm1i)
    y1c = jnp.clip(y0 + 1, 0, sm1i)
    x0 = jnp.clip(x0f.astype(jnp.int32), 0, sm1i)
    x1c = jnp.clip(x0 + 1, 0, sm1i)
    wy = ys - y0f
    wx = xs - x0f
    omy = 1.0 - wy
    omx = 1.0 - wx

    idx_ref[:, 0 * SPAD:1 * SPAD] = (off + y0 * side + x0)[:, :SPAD]
    idx_ref[:, 1 * SPAD:2 * SPAD] = (off + y0 * side + x1c)[:, :SPAD]
    idx_ref[:, 2 * SPAD:3 * SPAD] = (off + y1c * side + x0)[:, :SPAD]
    idx_ref[:, 3 * SPAD:4 * SPAD] = (off + y1c * side + x1c)[:, :SPAD]
    wts_ref[:, 0 * WPAD:1 * WPAD] = omy * omx
    wts_ref[:, 1 * WPAD:2 * WPAD] = omy * wx
    wts_ref[:, 2 * WPAD:3 * WPAD] = wy * omx
    wts_ref[:, 3 * WPAD:4 * WPAD] = wy * wx


def _prelude(boxesp, meta):
    return pl.pallas_call(
        _prelude_body,
        out_shape=[
            jax.ShapeDtypeStruct((NPAD, 4 * SPAD), jnp.int32),
            jax.ShapeDtypeStruct((NPAD, 4 * WPAD), jnp.float32),
        ],
    )(boxesp, meta)


_GD = lax.GatherDimensionNumbers(offset_dims=(), collapsed_slice_dims=(0,),
                                 start_index_map=(0,))


def _splat(vec, lane):
    return lax.gather(vec, jnp.full((16, 1), lane, jnp.int32), _GD,
                      slice_sizes=(1,),
                      mode=lax.GatherScatterMode.PROMISE_IN_BOUNDS)


def _combine(wts_v, r00, r01, r10, r11, o):
    def group(g, n_s):
        w00v = wts_v[pl.ds(0 * WPAD + g * 16, 16)]
        w01v = wts_v[pl.ds(1 * WPAD + g * 16, 16)]
        w10v = wts_v[pl.ds(2 * WPAD + g * 16, 16)]
        w11v = wts_v[pl.ds(3 * WPAD + g * 16, 16)]
        for sl_i in range(n_s):
            si = g * 16 + sl_i
            w00 = _splat(w00v, sl_i)
            w01 = _splat(w01v, sl_i)
            w10 = _splat(w10v, sl_i)
            w11 = _splat(w11v, sl_i)

            def ch_body(co, c2, si=si, w00=w00, w01=w01, w10=w10, w11=w11):
                for k in range(4):
                    sl = pl.ds(co * 64 + k * 16, 16)
                    acc = (r00[si, sl] * w00 + r01[si, sl] * w01
                           + r10[si, sl] * w10 + r11[si, sl] * w11)
                    o[pl.ds(si * C + co * 64 + k * 16, 16)] = acc
                return c2

            lax.fori_loop(0, C // 64, ch_body, 0)

    def g_body(g, c):
        group(g, 16)
        return c

    lax.fori_loop(0, NSAMP // 16, g_body, 0)
    group(NSAMP // 16, NSAMP - 16 * (NSAMP // 16))


def _gathers(table, idx_v, bufs, sem):
    return [pltpu.async_copy(table.at[idx_v.at[pl.ds(c * SPAD, SPAD)]],
                             bufs[c], sem) for c in range(4)]


OROW = NSAMP * C


def _sc_body(idx_hbm, wts_hbm, table, out_hbm,
             i_a, w_a, i_b, w_b, a00, a01, a10, a11, b00, b01, b10, b11,
             o, sem_ga, sem_gb, sem_o):
    wid = lax.axis_index("s") * 2 + lax.axis_index("c")
    base = wid * PER_TILE
    abufs = (a00, a01, a10, a11)
    bbufs = (b00, b01, b10, b11)

    def wait4(sem, bufs):
        for c in range(4):
            pltpu.make_async_copy(table.at[pl.ds(0, SPAD)], bufs[c],
                                  sem).wait()

    def wait_o():
        pltpu.make_async_copy(o, out_hbm.at[pl.ds(0, OROW)], sem_o).wait()

    pltpu.sync_copy(idx_hbm.at[base], i_a)
    pltpu.sync_copy(wts_hbm.at[base], w_a)
    _gathers(table, i_a, abufs, sem_ga)

    def pair_body(j, carry):
        box0 = base + 2 * j
        box1 = base + 2 * j + 1

        pltpu.sync_copy(idx_hbm.at[box1], i_b)
        pltpu.sync_copy(wts_hbm.at[box1], w_b)
        _gathers(table, i_b, bbufs, sem_gb)

        wait4(sem_ga, abufs)

        @pl.when((j > 0) & (box1 - 2 < NBOX))
        def _():
            wait_o()
        _combine(w_a, a00, a01, a10, a11, o)

        @pl.when(box0 < NBOX)
        def _():
            pltpu.async_copy(o, out_hbm.at[pl.ds(box0 * OROW, OROW)],
                             sem_o)

        @pl.when(j < PER_TILE // 2 - 1)
        def _():
            pltpu.sync_copy(idx_hbm.at[box0 + 2], i_a)
            pltpu.sync_copy(wts_hbm.at[box0 + 2], w_a)
            _gathers(table, i_a, abufs, sem_ga)

        wait4(sem_gb, bbufs)

        @pl.when(box0 < NBOX)
        def _():
            wait_o()
        _combine(w_b, b00, b01, b10, b11, o)

        @pl.when(box1 < NBOX)
        def _():
            pltpu.async_copy(o, out_hbm.at[pl.ds(box1 * OROW, OROW)],
                             sem_o)

        return carry

    lax.fori_loop(0, PER_TILE // 2, pair_body, 0)

    @pl.when(base + PER_TILE - 1 < NBOX)
    def _():
        wait_o()


@functools.cache
def _sc_call():
    return functools.partial(
        pl.kernel,
        mesh=plsc.VectorSubcoreMesh(core_axis_name="c", subcore_axis_name="s"),
        out_type=jax.ShapeDtypeStruct((NBOX * NSAMP * C,), jnp.float32),
        scratch_types=[
            pltpu.VMEM((4 * SPAD,), jnp.int32),
            pltpu.VMEM((4 * WPAD,), jnp.float32),
            pltpu.VMEM((4 * SPAD,), jnp.int32),
            pltpu.VMEM((4 * WPAD,), jnp.float32),
            pltpu.VMEM((SPAD, C), jnp.float32),
            pltpu.VMEM((SPAD, C), jnp.float32),
            pltpu.VMEM((SPAD, C), jnp.float32),
            pltpu.VMEM((SPAD, C), jnp.float32),
            pltpu.VMEM((SPAD, C), jnp.float32),
            pltpu.VMEM((SPAD, C), jnp.float32),
            pltpu.VMEM((SPAD, C), jnp.float32),
            pltpu.VMEM((SPAD, C), jnp.float32),
            pltpu.VMEM((NSAMP * C,), jnp.float32),
            pltpu.SemaphoreType.DMA,
            pltpu.SemaphoreType.DMA,
            pltpu.SemaphoreType.DMA,
        ],
    )(_sc_body)


def _copy_body(src_ref, dst_ref):
    dst_ref[...] = src_ref[...]


def _alias_copy_body(tbl_ref, src_ref, dst_ref):
    del tbl_ref
    dst_ref[...] = src_ref[...]


_CPB = 1024


def _build_table(p2, p3, p4, p5):
    rows = TROWS
    t = pl.pallas_call(
        _copy_body,
        grid=(p2.shape[0] // _CPB,),
        in_specs=[pl.BlockSpec((_CPB, C), lambda g: (g, 0))],
        out_specs=pl.BlockSpec((_CPB, C), lambda g: (g, 0)),
        out_shape=jax.ShapeDtypeStruct((rows, C), jnp.float32),
    )(p2)
    for pk, off in ((p3, OFF3), (p4, OFF4), (p5, OFF5)):
        t = pl.pallas_call(
            _alias_copy_body,
            grid=(pk.shape[0] // _CPB,) if pk.shape[0] >= _CPB else (1,),
            in_specs=[
                pl.BlockSpec(memory_space=pl.ANY),
                pl.BlockSpec((min(_CPB, pk.shape[0]), C), lambda g: (g, 0)),
            ],
            out_specs=pl.BlockSpec(
                (min(_CPB, pk.shape[0]), C),
                lambda g, off=off, n=min(_CPB, pk.shape[0]): (off // n + g, 0)),
            out_shape=jax.ShapeDtypeStruct((rows, C), jnp.float32),
            input_output_aliases={0: 0},
        )(t, pk)
    return t


def kernel(boxes, image_meta, p2, p3, p4, p5):
    boxes2 = boxes.reshape(-1, 4)
    boxesp = jnp.pad(boxes2, ((0, NPAD - boxes2.shape[0]), (0, 0)))
    idx, wts = _prelude(boxesp, image_meta)
    table = _build_table(p2.reshape(-1, C), p3.reshape(-1, C),
                         p4.reshape(-1, C), p5.reshape(-1, C))
    out = _sc_call()(idx, wts, table)
    return out.reshape(1, NBOX, POOL_H, POOL_W, C)


# speedup vs baseline: 1.6416x; 1.6416x over previous
import functools

import jax
import jax.numpy as jnp
from jax import lax
from jax.experimental import pallas as pl
from jax.experimental.pallas import tpu as pltpu
from jax.experimental.pallas import tpu_sc as plsc

POOL_H = 7
POOL_W = 7
NSAMP = POOL_H * POOL_W
SPAD = 56
WPAD = 64
NBOX = 1000
NPAD = 1024
NTILES = 32
PER_TILE = NPAD // NTILES
C = 256


OFF3 = 256 * 256
OFF4 = OFF3 + 128 * 128
OFF5 = OFF4 + 64 * 64
TROWS = OFF5 + 32 * 32


def _prelude_body(boxes_ref, imeta_ref, idx_ref, wts_ref):
    b = boxes_ref[...]
    y1 = b[:, 0:1]
    x1 = b[:, 1:2]
    y2 = b[:, 2:3]
    x2 = b[:, 3:4]
    h = y2 - y1
    w = x2 - x1
    m = imeta_ref[...]
    area = m[0, 4] * m[0, 5]
    rl = jnp.log(jnp.sqrt(h * w) / (224.0 / jnp.sqrt(area))) / jnp.log(2.0)
    lvl = jnp.minimum(5, jnp.maximum(2, 4 + jnp.round(rl).astype(jnp.int32)))
    side = jnp.right_shift(1024, lvl)
    off = jnp.where(lvl == 2, 0,
                    jnp.where(lvl == 3, OFF3,
                              jnp.where(lvl == 4, OFF4, OFF5)))
    sm1i = side - 1
    sm1f = sm1i.astype(jnp.float32)

    s = lax.broadcasted_iota(jnp.int32, (1, WPAD), 1)
    iy = (s // POOL_W).astype(jnp.float32)
    ix = (s % POOL_W).astype(jnp.float32)
    ys = y1 * sm1f + iy * (h * sm1f / (POOL_H - 1))
    xs = x1 * sm1f + ix * (w * sm1f / (POOL_W - 1))
    y0f = jnp.floor(ys)
    x0f = jnp.floor(xs)
    y0 = jnp.clip(y0f.astype(jnp.int32), 0, sm1i)
    y1c = jnp.clip(y0 + 1, 0, sm1i)
    x0 = jnp.clip(x0f.astype(jnp.int32), 0, sm1i)
    x1c = jnp.clip(x0 + 1, 0, sm1i)
    wy = ys - y0f
    wx = xs - x0f
    omy = 1.0 - wy
    omx = 1.0 - wx

    idx_ref[:, 0 * SPAD:1 * SPAD] = (off + y0 * side + x0)[:, :SPAD]
    idx_ref[:, 1 * SPAD:2 * SPAD] = (off + y0 * side + x1c)[:, :SPAD]
    idx_ref[:, 2 * SPAD:3 * SPAD] = (off + y1c * side + x0)[:, :SPAD]
    idx_ref[:, 3 * SPAD:4 * SPAD] = (off + y1c * side + x1c)[:, :SPAD]
    wts_ref[:, 0 * WPAD:1 * WPAD] = omy * omx
    wts_ref[:, 1 * WPAD:2 * WPAD] = omy * wx
    wts_ref[:, 2 * WPAD:3 * WPAD] = wy * omx
    wts_ref[:, 3 * WPAD:4 * WPAD] = wy * wx


def _prelude(boxesp, meta):
    return pl.pallas_call(
        _prelude_body,
        out_shape=[
            jax.ShapeDtypeStruct((NPAD, 4 * SPAD), jnp.int32),
            jax.ShapeDtypeStruct((NPAD, 4 * WPAD), jnp.float32),
        ],
    )(boxesp, meta)


_GD = lax.GatherDimensionNumbers(offset_dims=(), collapsed_slice_dims=(0,),
                                 start_index_map=(0,))


def _splat(vec, lane):
    return lax.gather(vec, jnp.full((16, 1), lane, jnp.int32), _GD,
                      slice_sizes=(1,),
                      mode=lax.GatherScatterMode.PROMISE_IN_BOUNDS)


def _combine(wts_v, r00, r01, r10, r11, o):
    def group(g, n_s):
        w00v = wts_v[pl.ds(0 * WPAD + g * 16, 16)]
        w01v = wts_v[pl.ds(1 * WPAD + g * 16, 16)]
        w10v = wts_v[pl.ds(2 * WPAD + g * 16, 16)]
        w11v = wts_v[pl.ds(3 * WPAD + g * 16, 16)]
        for sl_i in range(n_s):
            si = g * 16 + sl_i
            w00 = _splat(w00v, sl_i)
            w01 = _splat(w01v, sl_i)
            w10 = _splat(w10v, sl_i)
            w11 = _splat(w11v, sl_i)

            def ch_body(co, c2, si=si, w00=w00, w01=w01, w10=w10, w11=w11):
                for k in range(4):
                    sl = pl.ds(co * 64 + k * 16, 16)
                    acc = (r00[si, sl] * w00 + r01[si, sl] * w01
                           + r10[si, sl] * w10 + r11[si, sl] * w11)
                    o[si, sl] = acc
                return c2

            lax.fori_loop(0, C // 64, ch_body, 0)

    def g_body(g, c):
        group(g, 16)
        return c

    lax.fori_loop(0, NSAMP // 16, g_body, 0)
    group(NSAMP // 16, NSAMP - 16 * (NSAMP // 16))


def _gathers(table, idx_v, bufs, sem):
    return [pltpu.async_copy(table.at[idx_v.at[pl.ds(c * SPAD, SPAD)]],
                             bufs[c], sem) for c in range(4)]


def _sc_body(idx_hbm, wts_hbm, table, out_hbm,
             i_a, w_a, i_b, w_b, a00, a01, a10, a11, b00, b01, b10, b11,
             o, sem_ga, sem_gb, sem_o):
    wid = lax.axis_index("s") * 2 + lax.axis_index("c")
    base = wid * PER_TILE
    abufs = (a00, a01, a10, a11)
    bbufs = (b00, b01, b10, b11)

    def wait4(sem, bufs):
        for c in range(4):
            pltpu.make_async_copy(table.at[pl.ds(0, SPAD)], bufs[c],
                                  sem).wait()

    def wait_o():
        pltpu.make_async_copy(o, out_hbm.at[0], sem_o).wait()

    pltpu.sync_copy(idx_hbm.at[base], i_a)
    pltpu.sync_copy(wts_hbm.at[base], w_a)
    _gathers(table, i_a, abufs, sem_ga)

    def pair_body(j, carry):
        box0 = base + 2 * j
        box1 = base + 2 * j + 1

        pltpu.sync_copy(idx_hbm.at[box1], i_b)
        pltpu.sync_copy(wts_hbm.at[box1], w_b)
        _gathers(table, i_b, bbufs, sem_gb)

        wait4(sem_ga, abufs)

        @pl.when((j > 0) & (box1 - 2 < NBOX))
        def _():
            wait_o()
        _combine(w_a, a00, a01, a10, a11, o)

        @pl.when(box0 < NBOX)
        def _():
            pltpu.async_copy(o, out_hbm.at[box0], sem_o)

        @pl.when(j < PER_TILE // 2 - 1)
        def _():
            pltpu.sync_copy(idx_hbm.at[box0 + 2], i_a)
            pltpu.sync_copy(wts_hbm.at[box0 + 2], w_a)
            _gathers(table, i_a, abufs, sem_ga)

        wait4(sem_gb, bbufs)

        @pl.when(box0 < NBOX)
        def _():
            wait_o()
        _combine(w_b, b00, b01, b10, b11, o)

        @pl.when(box1 < NBOX)
        def _():
            pltpu.async_copy(o, out_hbm.at[box1], sem_o)

        return carry

    lax.fori_loop(0, PER_TILE // 2, pair_body, 0)

    @pl.when(base + PER_TILE - 1 < NBOX)
    def _():
        wait_o()


@functools.cache
def _sc_call():
    return functools.partial(
        pl.kernel,
        mesh=plsc.VectorSubcoreMesh(core_axis_name="c", subcore_axis_name="s"),
        out_type=jax.ShapeDtypeStruct((NBOX, NSAMP, C), jnp.float32),
        scratch_types=[
            pltpu.VMEM((4 * SPAD,), jnp.int32),
            pltpu.VMEM((4 * WPAD,), jnp.float32),
            pltpu.VMEM((4 * SPAD,), jnp.int32),
            pltpu.VMEM((4 * WPAD,), jnp.float32),
            pltpu.VMEM((SPAD, C), jnp.float32),
            pltpu.VMEM((SPAD, C), jnp.float32),
            pltpu.VMEM((SPAD, C), jnp.float32),
            pltpu.VMEM((SPAD, C), jnp.float32),
            pltpu.VMEM((SPAD, C), jnp.float32),
            pltpu.VMEM((SPAD, C), jnp.float32),
            pltpu.VMEM((SPAD, C), jnp.float32),
            pltpu.VMEM((SPAD, C), jnp.float32),
            pltpu.VMEM((NSAMP, C), jnp.float32),
            pltpu.SemaphoreType.DMA,
            pltpu.SemaphoreType.DMA,
            pltpu.SemaphoreType.DMA,
        ],
    )(_sc_body)


def _copy_body(src_ref, dst_ref):
    dst_ref[...] = src_ref[...]


def _alias_copy_body(tbl_ref, src_ref, dst_ref):
    del tbl_ref
    dst_ref[...] = src_ref[...]


_CPB = 1024


def _build_table(p2, p3, p4, p5):
    rows = TROWS
    t = pl.pallas_call(
        _copy_body,
        grid=(p2.shape[0] // _CPB,),
        in_specs=[pl.BlockSpec((_CPB, C), lambda g: (g, 0))],
        out_specs=pl.BlockSpec((_CPB, C), lambda g: (g, 0)),
        out_shape=jax.ShapeDtypeStruct((rows, C), jnp.float32),
    )(p2)
    for pk, off in ((p3, OFF3), (p4, OFF4), (p5, OFF5)):
        t = pl.pallas_call(
            _alias_copy_body,
            grid=(pk.shape[0] // _CPB,) if pk.shape[0] >= _CPB else (1,),
            in_specs=[
                pl.BlockSpec(memory_space=pl.ANY),
                pl.BlockSpec((min(_CPB, pk.shape[0]), C), lambda g: (g, 0)),
            ],
            out_specs=pl.BlockSpec(
                (min(_CPB, pk.shape[0]), C),
                lambda g, off=off, n=min(_CPB, pk.shape[0]): (off // n + g, 0)),
            out_shape=jax.ShapeDtypeStruct((rows, C), jnp.float32),
            input_output_aliases={0: 0},
        )(t, pk)
    return t


def kernel(boxes, image_meta, p2, p3, p4, p5):
    boxes2 = boxes.reshape(-1, 4)
    boxesp = jnp.pad(boxes2, ((0, NPAD - boxes2.shape[0]), (0, 0)))
    idx, wts = _prelude(boxesp, image_meta)
    table = _build_table(p2.reshape(-1, C), p3.reshape(-1, C),
                         p4.reshape(-1, C), p5.reshape(-1, C))
    out = _sc_call()(idx, wts, table)
    return out.reshape(1, NBOX, POOL_H, POOL_W, C)


# speedup vs baseline: 2.1105x; 1.2856x over previous
import functools

import jax
import jax.numpy as jnp
from jax import lax
from jax.experimental import pallas as pl
from jax.experimental.pallas import tpu as pltpu
from jax.experimental.pallas import tpu_sc as plsc

POOL_H = 7
POOL_W = 7
NSAMP = POOL_H * POOL_W
SPAD = 56
WPAD = 64
NBOX = 1000
NPAD = 1024
NTILES = 32
PER_TILE = NPAD // NTILES
C = 256


OFF3 = 256 * 256
OFF4 = OFF3 + 128 * 128
OFF5 = OFF4 + 64 * 64
TROWS = OFF5 + 32 * 32


def _prelude_body(boxes_ref, imeta_ref, idx_ref, wts_ref):
    b = boxes_ref[...]
    y1 = b[:, 0:1]
    x1 = b[:, 1:2]
    y2 = b[:, 2:3]
    x2 = b[:, 3:4]
    h = y2 - y1
    w = x2 - x1
    m = imeta_ref[...]
    area = m[0, 4] * m[0, 5]
    rl = jnp.log(jnp.sqrt(h * w) / (224.0 / jnp.sqrt(area))) / jnp.log(2.0)
    lvl = jnp.minimum(5, jnp.maximum(2, 4 + jnp.round(rl).astype(jnp.int32)))
    side = jnp.right_shift(1024, lvl)
    off = jnp.where(lvl == 2, 0,
                    jnp.where(lvl == 3, OFF3,
                              jnp.where(lvl == 4, OFF4, OFF5)))
    sm1i = side - 1
    sm1f = sm1i.astype(jnp.float32)

    s = lax.broadcasted_iota(jnp.int32, (1, WPAD), 1)
    iy = (s // POOL_W).astype(jnp.float32)
    ix = (s % POOL_W).astype(jnp.float32)
    ys = y1 * sm1f + iy * (h * sm1f / (POOL_H - 1))
    xs = x1 * sm1f + ix * (w * sm1f / (POOL_W - 1))
    y0f = jnp.floor(ys)
    x0f = jnp.floor(xs)
    y0 = jnp.clip(y0f.astype(jnp.int32), 0, sm1i)
    y1c = jnp.clip(y0 + 1, 0, sm1i)
    x0 = jnp.clip(x0f.astype(jnp.int32), 0, sm1i)
    x1c = jnp.clip(x0 + 1, 0, sm1i)
    wy = ys - y0f
    wx = xs - x0f
    omy = 1.0 - wy
    omx = 1.0 - wx

    idx_ref[:, 0 * SPAD:1 * SPAD] = (off + y0 * side + x0)[:, :SPAD]
    idx_ref[:, 1 * SPAD:2 * SPAD] = (off + y0 * side + x1c)[:, :SPAD]
    idx_ref[:, 2 * SPAD:3 * SPAD] = (off + y1c * side + x0)[:, :SPAD]
    idx_ref[:, 3 * SPAD:4 * SPAD] = (off + y1c * side + x1c)[:, :SPAD]
    wts_ref[:, 0 * WPAD:1 * WPAD] = omy * omx
    wts_ref[:, 1 * WPAD:2 * WPAD] = omy * wx
    wts_ref[:, 2 * WPAD:3 * WPAD] = wy * omx
    wts_ref[:, 3 * WPAD:4 * WPAD] = wy * wx


def _prelude(boxesp, meta):
    return pl.pallas_call(
        _prelude_body,
        out_shape=[
            jax.ShapeDtypeStruct((NPAD, 4 * SPAD), jnp.int32),
            jax.ShapeDtypeStruct((NPAD, 4 * WPAD), jnp.float32),
        ],
    )(boxesp, meta)


_GD = lax.GatherDimensionNumbers(offset_dims=(), collapsed_slice_dims=(0,),
                                 start_index_map=(0,))


def _splat(vec, lane):
    return lax.gather(vec, jnp.full((16, 1), lane, jnp.int32), _GD,
                      slice_sizes=(1,),
                      mode=lax.GatherScatterMode.PROMISE_IN_BOUNDS)


def _combine(wts_v, r00, r01, r10, r11, o):
    def group(g, n_s):
        w00v = wts_v[pl.ds(0 * WPAD + g * 16, 16)]
        w01v = wts_v[pl.ds(1 * WPAD + g * 16, 16)]
        w10v = wts_v[pl.ds(2 * WPAD + g * 16, 16)]
        w11v = wts_v[pl.ds(3 * WPAD + g * 16, 16)]
        for sl_i in range(n_s):
            si = g * 16 + sl_i
            w00 = _splat(w00v, sl_i)
            w01 = _splat(w01v, sl_i)
            w10 = _splat(w10v, sl_i)
            w11 = _splat(w11v, sl_i)

            def ch_body(co, c2, si=si, w00=w00, w01=w01, w10=w10, w11=w11):
                for k in range(8):
                    sl = pl.ds(co * 128 + k * 16, 16)
                    o[si, sl] = ((r00[si, sl] * w00 + r01[si, sl] * w01)
                                 + (r10[si, sl] * w10 + r11[si, sl] * w11))
                return c2

            lax.fori_loop(0, C // 128, ch_body, 0)

    def g_body(g, c):
        group(g, 16)
        return c

    lax.fori_loop(0, NSAMP // 16, g_body, 0)
    group(NSAMP // 16, NSAMP - 16 * (NSAMP // 16))


def _gathers(table, idx_v, bufs, sem):
    return [pltpu.async_copy(table.at[idx_v.at[pl.ds(c * SPAD, SPAD)]],
                             bufs[c], sem) for c in range(4)]


def _sc_body(idx_hbm, wts_hbm, table, out_hbm,
             i_a, w_a, i_b, w_b, a00, a01, a10, a11, b00, b01, b10, b11,
             o, sem_ga, sem_gb, sem_o):
    wid = lax.axis_index("s") * 2 + lax.axis_index("c")
    base = wid * PER_TILE
    abufs = (a00, a01, a10, a11)
    bbufs = (b00, b01, b10, b11)

    def wait4(sem, bufs):
        for c in range(4):
            pltpu.make_async_copy(table.at[pl.ds(0, SPAD)], bufs[c],
                                  sem).wait()

    def wait_o():
        pltpu.make_async_copy(o, out_hbm.at[0], sem_o).wait()

    pltpu.sync_copy(idx_hbm.at[base], i_a)
    pltpu.sync_copy(wts_hbm.at[base], w_a)
    _gathers(table, i_a, abufs, sem_ga)

    def pair_body(j, carry):
        box0 = base + 2 * j
        box1 = base + 2 * j + 1

        pltpu.sync_copy(idx_hbm.at[box1], i_b)
        pltpu.sync_copy(wts_hbm.at[box1], w_b)
        _gathers(table, i_b, bbufs, sem_gb)

        wait4(sem_ga, abufs)

        @pl.when((j > 0) & (box1 - 2 < NBOX))
        def _():
            wait_o()
        _combine(w_a, a00, a01, a10, a11, o)

        @pl.when(box0 < NBOX)
        def _():
            pltpu.async_copy(o, out_hbm.at[box0], sem_o)

        @pl.when(j < PER_TILE // 2 - 1)
        def _():
            pltpu.sync_copy(idx_hbm.at[box0 + 2], i_a)
            pltpu.sync_copy(wts_hbm.at[box0 + 2], w_a)
            _gathers(table, i_a, abufs, sem_ga)

        wait4(sem_gb, bbufs)

        @pl.when(box0 < NBOX)
        def _():
            wait_o()
        _combine(w_b, b00, b01, b10, b11, o)

        @pl.when(box1 < NBOX)
        def _():
            pltpu.async_copy(o, out_hbm.at[box1], sem_o)

        return carry

    lax.fori_loop(0, PER_TILE // 2, pair_body, 0)

    @pl.when(base + PER_TILE - 1 < NBOX)
    def _():
        wait_o()


@functools.cache
def _sc_call():
    return functools.partial(
        pl.kernel,
        mesh=plsc.VectorSubcoreMesh(core_axis_name="c", subcore_axis_name="s"),
        out_type=jax.ShapeDtypeStruct((NBOX, NSAMP, C), jnp.float32),
        scratch_types=[
            pltpu.VMEM((4 * SPAD,), jnp.int32),
            pltpu.VMEM((4 * WPAD,), jnp.float32),
            pltpu.VMEM((4 * SPAD,), jnp.int32),
            pltpu.VMEM((4 * WPAD,), jnp.float32),
            pltpu.VMEM((SPAD, C), jnp.float32),
            pltpu.VMEM((SPAD, C), jnp.float32),
            pltpu.VMEM((SPAD, C), jnp.float32),
            pltpu.VMEM((SPAD, C), jnp.float32),
            pltpu.VMEM((SPAD, C), jnp.float32),
            pltpu.VMEM((SPAD, C), jnp.float32),
            pltpu.VMEM((SPAD, C), jnp.float32),
            pltpu.VMEM((SPAD, C), jnp.float32),
            pltpu.VMEM((NSAMP, C), jnp.float32),
            pltpu.SemaphoreType.DMA,
            pltpu.SemaphoreType.DMA,
            pltpu.SemaphoreType.DMA,
        ],
    )(_sc_body)


def _copy_body(src_ref, dst_ref):
    dst_ref[...] = src_ref[...]


def _alias_copy_body(tbl_ref, src_ref, dst_ref):
    del tbl_ref
    dst_ref[...] = src_ref[...]


_CPB = 1024


def _build_table(p2, p3, p4, p5):
    rows = TROWS
    t = pl.pallas_call(
        _copy_body,
        grid=(p2.shape[0] // _CPB,),
        in_specs=[pl.BlockSpec((_CPB, C), lambda g: (g, 0))],
        out_specs=pl.BlockSpec((_CPB, C), lambda g: (g, 0)),
        out_shape=jax.ShapeDtypeStruct((rows, C), jnp.float32),
    )(p2)
    for pk, off in ((p3, OFF3), (p4, OFF4), (p5, OFF5)):
        t = pl.pallas_call(
            _alias_copy_body,
            grid=(pk.shape[0] // _CPB,) if pk.shape[0] >= _CPB else (1,),
            in_specs=[
                pl.BlockSpec(memory_space=pl.ANY),
                pl.BlockSpec((min(_CPB, pk.shape[0]), C), lambda g: (g, 0)),
            ],
            out_specs=pl.BlockSpec(
                (min(_CPB, pk.shape[0]), C),
                lambda g, off=off, n=min(_CPB, pk.shape[0]): (off // n + g, 0)),
            out_shape=jax.ShapeDtypeStruct((rows, C), jnp.float32),
            input_output_aliases={0: 0},
        )(t, pk)
    return t


def kernel(boxes, image_meta, p2, p3, p4, p5):
    boxes2 = boxes.reshape(-1, 4)
    boxesp = jnp.pad(boxes2, ((0, NPAD - boxes2.shape[0]), (0, 0)))
    idx, wts = _prelude(boxesp, image_meta)
    table = _build_table(p2.reshape(-1, C), p3.reshape(-1, C),
                         p4.reshape(-1, C), p5.reshape(-1, C))
    out = _sc_call()(idx, wts, table)
    return out.reshape(1, NBOX, POOL_H, POOL_W, C)
